# baseline (speedup 1.0000x reference)
import functools

import jax
import jax.numpy as jnp
from jax import lax
from jax.experimental import pallas as pl
from jax.experimental.pallas import tpu as pltpu
from jax.experimental.pallas import tpu_sc as plsc

MAXLEN = 200
DIM = 64
LANES = 16
NC, NS = 2, 16
NW = NC * NS
NBUF = 2
ROUNDS = MAXLEN // NBUF
SUB = 8
TW = 128


def _pe_table():
    position = jnp.arange(MAXLEN, dtype=jnp.float32)[:, None]
    div_term = jnp.exp(
        jnp.arange(0, DIM, 2, dtype=jnp.float32) * (-jnp.log(10000.0) / DIM)
    )
    pe = jnp.zeros((MAXLEN, (DIM + 1) // 2 * 2), dtype=jnp.float32)
    pe = pe.at[:, 0::2].set(jnp.sin(position * div_term))
    pe = pe.at[:, 1::2].set(jnp.cos(position * div_term))
    return pe[:, :DIM]


def _make_repack(vocab):
    ntc = vocab // TW
    rem = vocab - ntc * TW
    base, extra = ntc // NW, ntc % NW
    mesh = plsc.VectorSubcoreMesh(core_axis_name="c", subcore_axis_name="s")

    @functools.partial(
        pl.kernel,
        out_type=jax.ShapeDtypeStruct((vocab * DIM // TW, TW), jnp.float32),
        mesh=mesh,
        scratch_types=[
            [pltpu.VMEM((DIM, TW), jnp.float32) for _ in range(NBUF)],
            [pltpu.VMEM((TW * DIM // TW, TW), jnp.float32) for _ in range(NBUF)],
            [pltpu.VMEM((rem * DIM // TW, TW), jnp.float32)],
            [pltpu.SemaphoreType.DMA for _ in range(NBUF)],
            [pltpu.SemaphoreType.DMA for _ in range(NBUF)],
        ],
        compiler_params=pltpu.CompilerParams(
            use_tc_tiling_on_sc=True, needs_layout_passes=False
        ),
    )
    def repack(wt_hbm, wtail_hbm, w128_hbm, stage, obuf, tailv, rsem, wsem):
        wid = lax.axis_index("s") * NC + lax.axis_index("c")
        start = wid * base + jnp.minimum(wid, extra)
        cnt = base + jnp.where(wid < extra, 1, 0)
        iota = lax.iota(jnp.int32, LANES)
        jvecs = [iota + u * LANES for u in range(DIM // LANES)]

        @pl.when(wid == 0)
        def _tail():
            pltpu.sync_copy(wtail_hbm, tailv[0])
            pltpu.sync_copy(
                tailv[0], w128_hbm.at[pl.ds(ntc * DIM, rem * DIM // TW)]
            )

        nrounds = lax.div(cnt + NBUF - 1, NBUF)

        @pl.loop(0, nrounds)
        def _round(g):
            for k in range(NBUF):
                b = g * NBUF + k

                @pl.when(b < cnt)
                def _():
                    @pl.when(g > 0)
                    def _():
                        pltpu.make_async_copy(
                            obuf[k], w128_hbm.at[pl.ds(0, DIM)], wsem[k]
                        ).wait()

                    pltpu.async_copy(
                        wt_hbm.at[:, pl.ds((start + b) * TW, TW)],
                        stage[k],
                        rsem[k],
                    )
            for k in range(NBUF):
                b = g * NBUF + k

                @pl.when(b < cnt)
                def _():
                    it = start + b
                    pltpu.make_async_copy(
                        wt_hbm.at[:, pl.ds(0, TW)], stage[k], rsem[k]
                    ).wait()

                    @pl.loop(0, TW, unroll=8)
                    def _row(l0):
                        csplat = jnp.full((LANES,), l0, jnp.int32)
                        p = lax.div(l0 * DIM, TW)
                        c0 = lax.rem(l0 * DIM, TW)
                        for u in range(DIM // LANES):
                            vals = plsc.load_gather(stage[k], [jvecs[u], csplat])
                            obuf[k][p, pl.ds(c0 + u * LANES, LANES)] = vals

                    pltpu.async_copy(
                        obuf[k], w128_hbm.at[pl.ds(it * DIM, DIM)], wsem[k]
                    )

        for k in range(NBUF):
            pltpu.make_async_copy(
                obuf[k], w128_hbm.at[pl.ds(0, DIM)], wsem[k]
            ).wait()

    return repack


def _make_gather(batch, vocab):
    bblk = batch // NW
    tt_n = MAXLEN // SUB
    gblk = bblk // LANES
    mesh = plsc.VectorSubcoreMesh(core_axis_name="c", subcore_axis_name="s")

    @functools.partial(
        pl.kernel,
        out_type=jax.ShapeDtypeStruct(
            (MAXLEN, DIM // SUB, NW, SUB, bblk), jnp.float32
        ),
        mesh=mesh,
        scratch_types=[
            pltpu.VMEM((tt_n, SUB, bblk), jnp.int32),
            pltpu.VMEM((MAXLEN * DIM,), jnp.float32),
            [pltpu.VMEM((bblk, DIM), jnp.float32) for _ in range(NBUF)],
            [pltpu.VMEM((DIM // SUB, SUB, bblk), jnp.float32) for _ in range(NBUF)],
            [pltpu.SemaphoreType.DMA for _ in range(NBUF)],
            [pltpu.SemaphoreType.DMA for _ in range(NBUF)],
        ],
        compiler_params=pltpu.CompilerParams(
            use_tc_tiling_on_sc=False, needs_layout_passes=False
        ),
    )
    def sc_kernel(xt_hbm, w_hbm, pe_hbm, out_hbm, idx_v, pe_v, rows, tbuf, gsem, ssem):
        wid = lax.axis_index("s") * NC + lax.axis_index("c")
        pltpu.sync_copy(xt_hbm.at[:, wid], idx_v)
        pltpu.sync_copy(pe_hbm, pe_v)
        lane = lax.iota(jnp.int32, LANES)
        rowv = [lane + q * LANES for q in range(gblk)]

        @pl.loop(0, ROUNDS)
        def _round(g):
            gdesc = []
            for k in range(NBUF):
                t = g * NBUF + k
                gdesc.append(
                    pltpu.async_copy(
                        w_hbm.at[idx_v.at[lax.div(t, SUB), lax.rem(t, SUB)]],
                        rows[k],
                        gsem[k],
                    )
                )
            for k in range(NBUF):
                t = g * NBUF + k
                gdesc[k].wait()

                @pl.when(g > 0)
                def _():
                    pltpu.make_async_copy(
                        tbuf[k], out_hbm.at[0, :, 0], ssem[k]
                    ).wait()

                pe_base = t * DIM
                for j in range(DIM):
                    psplat = plsc.load_gather(
                        pe_v, [jnp.full((LANES,), pe_base + j, jnp.int32)]
                    )
                    csplat = jnp.full((LANES,), j, jnp.int32)
                    for q in range(gblk):
                        vals = plsc.load_gather(rows[k], [rowv[q], csplat])
                        tbuf[k][j // SUB, j % SUB, pl.ds(q * LANES, LANES)] = (
                            vals + psplat
                        )

                pltpu.async_copy(tbuf[k], out_hbm.at[t, :, wid], ssem[k])

        for k in range(NBUF):
            pltpu.make_async_copy(tbuf[k], out_hbm.at[0, :, 0], ssem[k]).wait()

    return sc_kernel


def kernel(x, W):
    b, t = x.shape
    v = W.shape[0]
    pe = _pe_table()
    bblk = b // NW
    ntc = v // TW
    xt = x.reshape(NW, bblk, t // SUB, SUB).transpose(2, 0, 3, 1)
    wt = W.T
    wtail = W[ntc * TW :].reshape((v - ntc * TW) * DIM // TW, TW)
    w128 = _make_repack(v)(wt, wtail)
    wrm = w128.reshape(v, DIM)
    out5 = _make_gather(b, v)(xt, wrm, pe.reshape(-1))
    return out5.transpose(2, 4, 0, 1, 3).reshape(b, t, DIM)


# speedup vs baseline: 1.0045x; 1.0045x over previous
import functools

import jax
import jax.numpy as jnp
from jax import lax
from jax.experimental import pallas as pl
from jax.experimental.pallas import tpu as pltpu
from jax.experimental.pallas import tpu_sc as plsc

MAXLEN = 200
DIM = 64
LANES = 16
NC, NS = 2, 16
NW = NC * NS
NBUF = 2
ROUNDS = MAXLEN // NBUF
SUB = 8
TW = 128


def _pe_table():
    position = jnp.arange(MAXLEN, dtype=jnp.float32)[:, None]
    div_term = jnp.exp(
        jnp.arange(0, DIM, 2, dtype=jnp.float32) * (-jnp.log(10000.0) / DIM)
    )
    pe = jnp.zeros((MAXLEN, (DIM + 1) // 2 * 2), dtype=jnp.float32)
    pe = pe.at[:, 0::2].set(jnp.sin(position * div_term))
    pe = pe.at[:, 1::2].set(jnp.cos(position * div_term))
    return pe[:, :DIM]


def _make_repack(vocab):
    ntc = vocab // TW
    rem = vocab - ntc * TW
    base, extra = ntc // NW, ntc % NW
    mesh = plsc.VectorSubcoreMesh(core_axis_name="c", subcore_axis_name="s")

    @functools.partial(
        pl.kernel,
        out_type=jax.ShapeDtypeStruct((vocab * DIM // TW, TW), jnp.float32),
        mesh=mesh,
        scratch_types=[
            [pltpu.VMEM((DIM, TW), jnp.float32) for _ in range(NBUF)],
            [pltpu.VMEM((TW * DIM // TW, TW), jnp.float32) for _ in range(NBUF)],
            [pltpu.VMEM((rem * DIM // TW, TW), jnp.float32)],
            [pltpu.SemaphoreType.DMA for _ in range(NBUF)],
            [pltpu.SemaphoreType.DMA for _ in range(NBUF)],
        ],
        compiler_params=pltpu.CompilerParams(
            use_tc_tiling_on_sc=True,
            needs_layout_passes=False,
            disable_bounds_checks=True,
        ),
    )
    def repack(wt_hbm, wtail_hbm, w128_hbm, stage, obuf, tailv, rsem, wsem):
        wid = lax.axis_index("s") * NC + lax.axis_index("c")
        start = wid * base + jnp.minimum(wid, extra)
        cnt = base + jnp.where(wid < extra, 1, 0)
        iota = lax.iota(jnp.int32, LANES)
        jvecs = [iota + u * LANES for u in range(DIM // LANES)]

        @pl.when(wid == 0)
        def _tail():
            pltpu.sync_copy(wtail_hbm, tailv[0])
            pltpu.sync_copy(
                tailv[0], w128_hbm.at[pl.ds(ntc * DIM, rem * DIM // TW)]
            )

        nrounds = lax.div(cnt + NBUF - 1, NBUF)

        @pl.loop(0, nrounds)
        def _round(g):
            for k in range(NBUF):
                b = g * NBUF + k

                @pl.when(b < cnt)
                def _():
                    @pl.when(g > 0)
                    def _():
                        pltpu.make_async_copy(
                            obuf[k], w128_hbm.at[pl.ds(0, DIM)], wsem[k]
                        ).wait()

                    pltpu.async_copy(
                        wt_hbm.at[:, pl.ds((start + b) * TW, TW)],
                        stage[k],
                        rsem[k],
                    )
            for k in range(NBUF):
                b = g * NBUF + k

                @pl.when(b < cnt)
                def _():
                    it = start + b
                    pltpu.make_async_copy(
                        wt_hbm.at[:, pl.ds(0, TW)], stage[k], rsem[k]
                    ).wait()

                    @pl.loop(0, TW, unroll=8)
                    def _row(l0):
                        csplat = jnp.full((LANES,), l0, jnp.int32)
                        p = lax.div(l0 * DIM, TW)
                        c0 = lax.rem(l0 * DIM, TW)
                        for u in range(DIM // LANES):
                            vals = plsc.load_gather(stage[k], [jvecs[u], csplat])
                            obuf[k][p, pl.ds(c0 + u * LANES, LANES)] = vals

                    pltpu.async_copy(
                        obuf[k], w128_hbm.at[pl.ds(it * DIM, DIM)], wsem[k]
                    )

        for k in range(NBUF):
            pltpu.make_async_copy(
                obuf[k], w128_hbm.at[pl.ds(0, DIM)], wsem[k]
            ).wait()

    return repack


def _make_gather(batch, vocab):
    bblk = batch // NW
    tt_n = MAXLEN // SUB
    gblk = bblk // LANES
    mesh = plsc.VectorSubcoreMesh(core_axis_name="c", subcore_axis_name="s")

    @functools.partial(
        pl.kernel,
        out_type=jax.ShapeDtypeStruct(
            (MAXLEN, DIM // SUB, NW, SUB, bblk), jnp.float32
        ),
        mesh=mesh,
        scratch_types=[
            pltpu.VMEM((tt_n, SUB, bblk), jnp.int32),
            pltpu.VMEM((MAXLEN * DIM,), jnp.float32),
            [pltpu.VMEM((bblk, DIM), jnp.float32) for _ in range(NBUF)],
            [pltpu.VMEM((DIM // SUB, SUB, bblk), jnp.float32) for _ in range(NBUF)],
            [pltpu.SemaphoreType.DMA for _ in range(NBUF)],
            [pltpu.SemaphoreType.DMA for _ in range(NBUF)],
        ],
        compiler_params=pltpu.CompilerParams(
            use_tc_tiling_on_sc=False,
            needs_layout_passes=False,
            disable_bounds_checks=True,
        ),
    )
    def sc_kernel(xt_hbm, w_hbm, pe_hbm, out_hbm, idx_v, pe_v, rows, tbuf, gsem, ssem):
        wid = lax.axis_index("s") * NC + lax.axis_index("c")
        pltpu.sync_copy(xt_hbm.at[:, wid], idx_v)
        pltpu.sync_copy(pe_hbm, pe_v)
        lane = lax.iota(jnp.int32, LANES)
        rowv = [lane + q * LANES for q in range(gblk)]

        @pl.loop(0, ROUNDS)
        def _round(g):
            gdesc = []
            for k in range(NBUF):
                t = g * NBUF + k
                gdesc.append(
                    pltpu.async_copy(
                        w_hbm.at[idx_v.at[lax.div(t, SUB), lax.rem(t, SUB)]],
                        rows[k],
                        gsem[k],
                    )
                )
            for k in range(NBUF):
                t = g * NBUF + k
                gdesc[k].wait()

                @pl.when(g > 0)
                def _():
                    pltpu.make_async_copy(
                        tbuf[k], out_hbm.at[0, :, 0], ssem[k]
                    ).wait()

                pe_base = t * DIM
                for j in range(DIM):
                    psplat = plsc.load_gather(
                        pe_v, [jnp.full((LANES,), pe_base + j, jnp.int32)]
                    )
                    csplat = jnp.full((LANES,), j, jnp.int32)
                    for q in range(gblk):
                        vals = plsc.load_gather(rows[k], [rowv[q], csplat])
                        tbuf[k][j // SUB, j % SUB, pl.ds(q * LANES, LANES)] = (
                            vals + psplat
                        )

                pltpu.async_copy(tbuf[k], out_hbm.at[t, :, wid], ssem[k])

        for k in range(NBUF):
            pltpu.make_async_copy(tbuf[k], out_hbm.at[0, :, 0], ssem[k]).wait()

    return sc_kernel


def kernel(x, W):
    b, t = x.shape
    v = W.shape[0]
    pe = _pe_table()
    bblk = b // NW
    ntc = v // TW
    xt = x.reshape(NW, bblk, t // SUB, SUB).transpose(2, 0, 3, 1)
    wt = W.T
    wtail = W[ntc * TW :].reshape((v - ntc * TW) * DIM // TW, TW)
    w128 = _make_repack(v)(wt, wtail)
    wrm = w128.reshape(v, DIM)
    out5 = _make_gather(b, v)(xt, wrm, pe.reshape(-1))
    return out5.transpose(2, 4, 0, 1, 3).reshape(b, t, DIM)
